# Initial kernel scaffold; baseline (speedup 1.0000x reference)
import jax
import jax.numpy as jnp
from jax.experimental import pallas as pl


def kernel(input, embedding_weight):
    raise NotImplementedError("write your pallas kernel here")


# speedup vs baseline: 1.1067x; 1.1067x over previous
import functools

import jax
import jax.numpy as jnp
from jax import lax
from jax.experimental import pallas as pl
from jax.experimental.pallas import tpu as pltpu
from jax.experimental.pallas import tpu_sc as plsc

_NC = 2
_NS = 16
_NW = _NC * _NS


@functools.lru_cache(maxsize=None)
def _make_gather(V, D, B):
    assert B % _NW == 0
    b_per_w = B // _NW
    chunk = 1600
    assert b_per_w % chunk == 0
    n_chunks = b_per_w // chunk

    mesh = plsc.VectorSubcoreMesh(core_axis_name="c", subcore_axis_name="s")

    @functools.partial(
        pl.kernel,
        out_type=jax.ShapeDtypeStruct((B, D), jnp.float32),
        mesh=mesh,
        scratch_types=[
            pltpu.VMEM((b_per_w,), jnp.int32),
            pltpu.VMEM((chunk, D), jnp.float32),
            pltpu.SemaphoreType.DMA,
        ],
        compiler_params=pltpu.CompilerParams(use_tc_tiling_on_sc=False),
    )
    def k(idx_hbm, table_hbm, out_hbm, idx_v, rows_v, gsem):
        wid = lax.axis_index("s") * _NC + lax.axis_index("c")
        base = wid * b_per_w
        pltpu.sync_copy(idx_hbm.at[pl.ds(base, b_per_w)], idx_v)

        def step(g, carry):
            off = g * chunk
            pltpu.async_copy(
                table_hbm.at[idx_v.at[pl.ds(off, chunk)]], rows_v, gsem
            ).wait()
            pltpu.sync_copy(rows_v, out_hbm.at[pl.ds(base + off, chunk)])
            return carry

        lax.fori_loop(0, n_chunks, step, 0)

    return k


def kernel(input, embedding_weight):
    V, D = embedding_weight.shape
    flat = input.reshape(-1).astype(jnp.int32)
    out = _make_gather(V, D, flat.shape[0])(flat, embedding_weight)
    return out.reshape(*input.shape, D)


# speedup vs baseline: 1.1133x; 1.0060x over previous
import functools

import jax
import jax.numpy as jnp
from jax import lax
from jax.experimental import pallas as pl
from jax.experimental.pallas import tpu as pltpu
from jax.experimental.pallas import tpu_sc as plsc

_NC = 2
_NS = 16
_NW = _NC * _NS

_NBUF = 8
_LOOKAHEAD = 4


@functools.lru_cache(maxsize=None)
def _make_gather(V, D, B, chunk):
    assert B % _NW == 0
    b_per_w = B // _NW
    assert b_per_w % chunk == 0
    n_chunks = b_per_w // chunk
    n_groups = n_chunks // _NBUF
    assert n_chunks % _NBUF == 0 and n_groups >= 2

    mesh = plsc.VectorSubcoreMesh(core_axis_name="c", subcore_axis_name="s")

    @functools.partial(
        pl.kernel,
        out_type=jax.ShapeDtypeStruct((B, D), jnp.float32),
        mesh=mesh,
        scratch_types=[
            pltpu.VMEM((b_per_w,), jnp.int32),
            [pltpu.VMEM((chunk, D), jnp.float32) for _ in range(_NBUF)],
            pltpu.SemaphoreType.DMA,
            pltpu.SemaphoreType.DMA,
        ],
        compiler_params=pltpu.CompilerParams(use_tc_tiling_on_sc=False),
    )
    def k(idx_hbm, table_hbm, out_hbm, idx_v, rows, gsem, wsem):
        wid = lax.axis_index("s") * _NC + lax.axis_index("c")
        base = wid * b_per_w
        pltpu.sync_copy(idx_hbm.at[pl.ds(base, b_per_w)], idx_v)

        def start_gather(b, g):
            pltpu.async_copy(
                table_hbm.at[idx_v.at[pl.ds(g * chunk, chunk)]], rows[b], gsem
            )

        def wait_gather(b):
            pltpu.make_async_copy(
                table_hbm.at[idx_v.at[pl.ds(0, chunk)]], rows[b], gsem
            ).wait()

        def start_writeback(b, g):
            pltpu.async_copy(
                rows[b], out_hbm.at[pl.ds(base + g * chunk, chunk)], wsem
            )

        def wait_writeback(b):
            pltpu.make_async_copy(
                rows[b], out_hbm.at[pl.ds(base, chunk)], wsem
            ).wait()

        def emit_iter(g, j, wb_wait, do_gather):
            if do_gather:
                if wb_wait:
                    wait_writeback((j + _LOOKAHEAD) % _NBUF)
                start_gather((j + _LOOKAHEAD) % _NBUF, g + _LOOKAHEAD)
            wait_gather(j)
            start_writeback(j, g)

        for j in range(_LOOKAHEAD):
            start_gather(j, j)
        for j in range(_NBUF):
            emit_iter(j, j, wb_wait=(j >= _LOOKAHEAD), do_gather=True)

        def group(G, carry):
            g0 = G * _NBUF
            for j in range(_NBUF):
                emit_iter(g0 + j, j, wb_wait=True, do_gather=True)
            return carry

        lax.fori_loop(1, n_groups - 1, group, 0)

        g0 = (n_groups - 1) * _NBUF
        for j in range(_NBUF):
            emit_iter(g0 + j, j, wb_wait=(j < _LOOKAHEAD),
                      do_gather=(j < _LOOKAHEAD))
        for j in range(_NBUF):
            wait_writeback(j)

    return k


def kernel(input, embedding_weight):
    V, D = embedding_weight.shape
    flat = input.reshape(-1).astype(jnp.int32)
    out = _make_gather(V, D, flat.shape[0], 320)(flat, embedding_weight)
    return out.reshape(*input.shape, D)


# speedup vs baseline: 1.4682x; 1.3188x over previous
import functools

import jax
import jax.numpy as jnp
from jax import lax
from jax.experimental import pallas as pl
from jax.experimental.pallas import tpu as pltpu
from jax.experimental.pallas import tpu_sc as plsc

_NC = 2
_NS = 16
_NW = _NC * _NS

_S = 50
_B = 16384
_D = 32
_ST_FULL = _S // 8
_S_TAIL = _S - 8 * _ST_FULL
_BT = _B // 128
_FULL_TILES = _ST_FULL * _BT
_TILES_PER_W = (_ST_FULL + 1) * _BT // _NW
_FULL_PER_W = _FULL_TILES // _NW


@functools.lru_cache(maxsize=None)
def _make_lookup(Vq):
    mesh = plsc.VectorSubcoreMesh(core_axis_name="c", subcore_axis_name="s")

    @functools.partial(
        pl.kernel,
        out_type=jax.ShapeDtypeStruct((_S, _D, _B), jnp.float32),
        mesh=mesh,
        scratch_types=[
            pltpu.VMEM((8, 128), jnp.int32),
            pltpu.VMEM((1024,), jnp.int32),
            pltpu.VMEM((1024,), jnp.int32),
            [pltpu.VMEM((256, 128), jnp.float32) for _ in range(2)],
            [pltpu.VMEM((_D, 128), jnp.float32) for _ in range(2)],
            pltpu.SemaphoreType.DMA,
            pltpu.SemaphoreType.DMA,
        ],
        compiler_params=pltpu.CompilerParams(
            use_tc_tiling_on_sc=True, needs_layout_passes=False),
    )
    def k(idxT, tabL, outT, idx_v, qv, cv, rows, ostage, gsem, wsem):
        wid = lax.axis_index("s") * _NC + lax.axis_index("c")

        def prep_indices(s_cnt):
            def body(kk, carry):
                s = kk // 8
                v = idx_v[s, pl.ds(16 * lax.rem(kk, 8), 16)]
                o = 16 * kk
                qv[pl.ds(o, 16)] = jax.lax.shift_right_logical(v, 2)
                cv[pl.ds(o, 16)] = jax.lax.shift_left(
                    jax.lax.bitwise_and(v, 3), 5)
                return carry

            lax.fori_loop(0, 8 * s_cnt, body, 0)

        def process_tile(st, bt, s_cnt):
            pltpu.sync_copy(
                idxT.at[pl.ds(8 * st, s_cnt), pl.ds(128 * bt, 128)],
                idx_v.at[pl.ds(0, s_cnt)])
            prep_indices(s_cnt)
            n_ch = (s_cnt * 128) // 256
            pltpu.async_copy(
                tabL.at[qv.at[pl.ds(0, 256)]], rows[0], gsem)
            for ch in range(n_ch):
                b = ch % 2
                pltpu.make_async_copy(
                    tabL.at[qv.at[pl.ds(0, 256)]], rows[b], gsem).wait()
                if ch + 1 < n_ch:
                    pltpu.async_copy(
                        tabL.at[qv.at[pl.ds(256 * (ch + 1), 256)]],
                        rows[1 - b], gsem)
                for s_in in range(2):
                    s_loc = 2 * ch + s_in
                    sel_buf = b
                    obuf = ostage[s_in]

                    def bg_body(bg, carry, s_in=s_in, ch=ch,
                                sel_buf=sel_buf, obuf=obuf):
                        rvec = jax.lax.broadcasted_iota(
                            jnp.int32, (16,), 0) + (s_in * 128 + 16 * bg)
                        cbase = cv[pl.ds(256 * ch + s_in * 128 + 16 * bg,
                                         16)]
                        for d in range(_D):
                            val = plsc.load_gather(
                                rows[sel_buf], [rvec, cbase + d])
                            obuf[d, pl.ds(16 * bg, 16)] = val
                        return carry

                    lax.fori_loop(0, 8, bg_body, 0)
                    pltpu.async_copy(
                        obuf,
                        outT.at[8 * st + s_loc, :, pl.ds(128 * bt, 128)],
                        wsem)
                    pltpu.make_async_copy(
                        obuf,
                        outT.at[0, :, pl.ds(0, 128)],
                        wsem).wait()

        def full_body(i, carry):
            t = wid + _NW * i
            st = t // _BT
            bt = lax.rem(t, _BT)
            process_tile(st, bt, 8)
            return carry

        lax.fori_loop(0, _FULL_PER_W, full_body, 0)

        def tail_body(i, carry):
            bt = wid + _NW * i
            process_tile(_ST_FULL, bt, _S_TAIL)
            return carry

        lax.fori_loop(0, _TILES_PER_W - _FULL_PER_W, tail_body, 0)

    return k


def kernel(input, embedding_weight):
    V, D = embedding_weight.shape
    idxT = input.T
    tabL = embedding_weight.reshape(-1, 128)
    outT = _make_lookup(tabL.shape[0])(idxT, tabL)
    return outT.transpose(2, 0, 1)


# speedup vs baseline: 1.5583x; 1.0614x over previous
import functools

import jax
import jax.numpy as jnp
from jax import lax
from jax.experimental import pallas as pl
from jax.experimental.pallas import tpu as pltpu
from jax.experimental.pallas import tpu_sc as plsc

_NC = 2
_NS = 16
_NW = _NC * _NS

_S = 50
_B = 16384
_D = 32
_ST_FULL = _S // 8
_S_TAIL = _S - 8 * _ST_FULL
_BT = _B // 128
_FULL_TILES = _ST_FULL * _BT
_TILES_PER_W = (_ST_FULL + 1) * _BT // _NW
_FULL_PER_W = _FULL_TILES // _NW
_TAIL_PER_W = _TILES_PER_W - _FULL_PER_W
_NCH = 8 * _FULL_PER_W + _S_TAIL * _TAIL_PER_W
_NFCH = 8 * _FULL_PER_W


@functools.lru_cache(maxsize=None)
def _make_lookup(Vq):
    mesh = plsc.VectorSubcoreMesh(core_axis_name="c", subcore_axis_name="s")

    @functools.partial(
        pl.kernel,
        out_type=jax.ShapeDtypeStruct((_S, _D, _B), jnp.float32),
        mesh=mesh,
        scratch_types=[
            pltpu.VMEM((_TILES_PER_W, 8, 128), jnp.int32),
            pltpu.VMEM((4, 128), jnp.int32),
            pltpu.VMEM((4, 128, 128), jnp.float32),
            [pltpu.VMEM((_D, 128), jnp.float32) for _ in range(2)],
            pltpu.SemaphoreType.DMA,
            pltpu.SemaphoreType.DMA,
            pltpu.SemaphoreType.DMA,
        ],
        compiler_params=pltpu.CompilerParams(
            use_tc_tiling_on_sc=True, needs_layout_passes=False),
    )
    def k(idxT, tabL, outT, idx_all, qring, rows, ostage, isem, gsem, wsem):
        wid = lax.axis_index("s") * _NC + lax.axis_index("c")

        def stage_full(kt, carry):
            t = wid + _NW * kt
            st = jax.lax.shift_right_logical(t, 7)
            bt = jax.lax.bitwise_and(t, _BT - 1)
            pltpu.async_copy(
                idxT.at[pl.ds(8 * st, 8), pl.ds(128 * bt, 128)],
                idx_all.at[kt], isem)
            return carry

        lax.fori_loop(0, _FULL_PER_W, stage_full, 0)

        def stage_tail(i, carry):
            bt = wid + _NW * i
            pltpu.async_copy(
                idxT.at[pl.ds(8 * _ST_FULL, _S_TAIL), pl.ds(128 * bt, 128)],
                idx_all.at[_FULL_PER_W + i, pl.ds(0, _S_TAIL)], isem)
            return carry

        lax.fori_loop(0, _TAIL_PER_W, stage_tail, 0)

        def drain_full(i, carry):
            pltpu.make_async_copy(
                idxT.at[pl.ds(0, 8), pl.ds(0, 128)],
                idx_all.at[0], isem).wait()
            return carry

        lax.fori_loop(0, _FULL_PER_W, drain_full, 0)

        def drain_tail(i, carry):
            pltpu.make_async_copy(
                idxT.at[pl.ds(0, _S_TAIL), pl.ds(0, 128)],
                idx_all.at[0, pl.ds(0, _S_TAIL)], isem).wait()
            return carry

        lax.fori_loop(0, _TAIL_PER_W, drain_tail, 0)

        def chunk_coords(c):
            full = c < _NFCH
            k_f = jax.lax.shift_right_logical(c, 3)
            k_t = _FULL_PER_W + jax.lax.shift_right_logical(c - _NFCH, 1)
            kt = jnp.where(full, k_f, k_t)
            s_loc = jnp.where(full, jax.lax.bitwise_and(c, 7),
                              jax.lax.bitwise_and(c - _NFCH, 1))
            t = wid + _NW * kt
            st_f = jax.lax.shift_right_logical(t, 7)
            st = jnp.where(full, st_f, _ST_FULL)
            bt = jnp.where(full, jax.lax.bitwise_and(t, _BT - 1),
                           wid + _NW * (kt - _FULL_PER_W))
            return kt, s_loc, 8 * st + s_loc, bt

        def prep_and_fire(c, slot):
            kt, s_loc, _, _ = chunk_coords(c)
            for g in range(8):
                v = idx_all[kt, s_loc, pl.ds(16 * g, 16)]
                qring[slot, pl.ds(16 * g, 16)] = (
                    jax.lax.shift_right_logical(v, 2))
            pltpu.async_copy(tabL.at[qring.at[slot]], rows.at[slot], gsem)

        def wait_gather(slot):
            pltpu.make_async_copy(
                tabL.at[qring.at[slot]], rows.at[slot], gsem).wait()

        def select_and_out(c, slot, oslot):
            kt, s_loc, s_glob, bt = chunk_coords(c)
            obuf = ostage[oslot]
            for bg in range(8):
                rvec = jax.lax.broadcasted_iota(
                    jnp.int32, (16,), 0) + (16 * bg)
                cvec = jax.lax.shift_left(jax.lax.bitwise_and(
                    idx_all[kt, s_loc, pl.ds(16 * bg, 16)], 3), 5)
                for d in range(_D):
                    obuf[d, pl.ds(16 * bg, 16)] = plsc.load_gather(
                        rows.at[slot], [rvec, cvec + d])
            pltpu.async_copy(
                obuf, outT.at[s_glob, :, pl.ds(128 * bt, 128)], wsem)

        def wait_out(oslot):
            pltpu.make_async_copy(
                ostage[oslot], outT.at[0, :, pl.ds(0, 128)], wsem).wait()

        prep_and_fire(0, 0)
        prep_and_fire(1, 1)

        def group(G, carry):
            c0 = 4 * G
            for j in range(4):
                c = c0 + j

                @pl.when(c + 2 < _NCH)
                def _():
                    prep_and_fire(c + 2, (j + 2) % 4)

                wait_gather(j)

                @pl.when(c >= 2)
                def _():
                    wait_out(j % 2)

                select_and_out(c, j, j % 2)
            return carry

        lax.fori_loop(0, _NCH // 4, group, 0)
        wait_out(0)
        wait_out(1)

    return k


def kernel(input, embedding_weight):
    V, D = embedding_weight.shape
    idxT = input.T
    tabL = embedding_weight.reshape(-1, 128)
    outT = _make_lookup(tabL.shape[0])(idxT, tabL)
    return outT.transpose(2, 0, 1)
